# baseline (speedup 1.0000x reference)
import functools
import math

import jax
import jax.numpy as jnp
from jax import lax
from jax.experimental import pallas as pl
from jax.experimental.pallas import tpu as pltpu
from jax.experimental.pallas import tpu_sc as plsc

B = 2
P = 512
T = B * P
D_MODEL = 768
N_HEADS = 12
D_HEAD = D_MODEL // N_HEADS
D_FF = 3072
N_EXPERTS = 4
M_FEATS = int(D_HEAD * math.log(D_HEAD))
M_PAD = 384
HEADS_PER_BLK = 2
CIN_FLAT = 128
F_BLK = 1536
N_FBLK = D_FF // F_BLK
BT = 256
NBLK = 12
SROWS = NBLK * BT
NC, NS = 2, 16
NW = NC * NS


def _embed_qkv_body(xf_ref, wemb_ref, wq_ref, bq_ref, wk_ref, bk_ref,
                    wv_ref, bv_ref, h_ref, qn_ref, kn_ref, vn_ref):
    h = jnp.dot(xf_ref[...], wemb_ref[...], preferred_element_type=jnp.float32)
    h_ref[...] = h
    dn = 1.0 / (float(D_HEAD) ** 0.25)
    qn_ref[...] = (jnp.dot(h, wq_ref[...], preferred_element_type=jnp.float32)
                   + bq_ref[...]) * dn
    kn_ref[...] = (jnp.dot(h, wk_ref[...], preferred_element_type=jnp.float32)
                   + bk_ref[...]) * dn
    vn_ref[...] = jnp.dot(h, wv_ref[...], preferred_element_type=jnp.float32) + bv_ref[...]


def _performer_one_head(qb, kb, vb, om):
    dims = (((1,), (1,)), ((), ()))
    pq = jax.lax.dot_general(qb, om, dims, preferred_element_type=jnp.float32)
    pk = jax.lax.dot_general(kb, om, dims, preferred_element_type=jnp.float32)
    lane = jax.lax.broadcasted_iota(jnp.int32, (P, M_PAD), 1)
    valid = lane < M_FEATS
    dq = 0.5 * jnp.sum(qb * qb, axis=-1, keepdims=True)
    dk = 0.5 * jnp.sum(kb * kb, axis=-1, keepdims=True)
    mq = jnp.max(pq, axis=-1, keepdims=True)
    mk = jnp.max(pk)
    inv_sqrt_m = jnp.float32(1.0 / math.sqrt(M_FEATS))
    eps = jnp.float32(1e-6)
    qp = jnp.exp(pq - (dq + mq)) * inv_sqrt_m + eps
    kp = jnp.where(valid, jnp.exp(pk - (dk + mk)) * inv_sqrt_m + eps, 0.0)
    kv = jax.lax.dot_general(kp, vb, (((0,), (0,)), ((), ())),
                             preferred_element_type=jnp.float32)
    ksum = jnp.sum(kp, axis=0, keepdims=True)
    z = jax.lax.dot_general(qp, ksum, (((1,), (1,)), ((), ())),
                            preferred_element_type=jnp.float32)
    return jnp.dot(qp, kv, preferred_element_type=jnp.float32) / z


def _performer_body(qn_ref, kn_ref, vn_ref, om_ref, o_ref):
    om = om_ref[...]
    outs = []
    for hh in range(HEADS_PER_BLK):
        sl = (slice(None), slice(hh * D_HEAD, (hh + 1) * D_HEAD))
        outs.append(_performer_one_head(
            qn_ref[0][sl], kn_ref[0][sl], vn_ref[0][sl], om))
    o_ref[0] = jnp.concatenate(outs, axis=1)


def _layernorm(s, g, b):
    mu = jnp.mean(s, axis=-1, keepdims=True)
    var = jnp.mean((s - mu) ** 2, axis=-1, keepdims=True)
    return (s - mu) * jax.lax.rsqrt(var + 1e-5) * g + b


def _proj_router_body(attn_ref, wo_ref, bo_ref, h_ref, g2_ref, be2_ref,
                      wg_ref, bg_ref, h2_ref, route_ref, src_ref, meta_ref):
    a = jnp.dot(attn_ref[...], wo_ref[...], preferred_element_type=jnp.float32) + bo_ref[...]
    h2 = _layernorm(h_ref[...] + a, g2_ref[...], be2_ref[...])
    h2_ref[...] = h2

    logits = jnp.dot(h2, wg_ref[...], preferred_element_type=jnp.float32) + bg_ref[...]
    lane = jax.lax.broadcasted_iota(jnp.int32, (T, 128), 1)
    logits = jnp.where(lane < N_EXPERTS, logits, jnp.float32(-1e30))
    big = jnp.int32(10**6)
    m1 = jnp.max(logits, axis=-1, keepdims=True)
    i1 = jnp.min(jnp.where(logits == m1, lane, big), axis=-1, keepdims=True)
    first1 = lane == i1
    l2 = jnp.where(first1, jnp.float32(-1e30), logits)
    m2 = jnp.max(l2, axis=-1, keepdims=True)
    i2 = jnp.min(jnp.where(l2 == m2, lane, big), axis=-1, keepdims=True)
    first2 = lane == i2
    ex = jnp.exp(m2 - m1)
    g1 = 1.0 / (1.0 + ex)
    g2v = ex / (1.0 + ex)

    oh = (jnp.where(first1, 1.0, 0.0) + jnp.where(first2, 1.0, 0.0))
    r_i = jax.lax.broadcasted_iota(jnp.int32, (T, T), 0)
    c_i = jax.lax.broadcasted_iota(jnp.int32, (T, T), 1)
    ltri = jnp.where(c_i < r_i, 1.0, 0.0)
    pos = jnp.dot(ltri, oh, preferred_element_type=jnp.float32)
    cnt = jnp.sum(oh, axis=0, keepdims=True)
    cntp = jnp.ceil(cnt * (1.0 / BT)) * BT
    u_i = jax.lax.broadcasted_iota(jnp.int32, (128, 128), 0)
    v_i = jax.lax.broadcasted_iota(jnp.int32, (128, 128), 1)
    ut = jnp.where(u_i < v_i, 1.0, 0.0)
    base = jnp.dot(cntp, ut, preferred_element_type=jnp.float32)
    dest = base + pos
    d0 = jnp.sum(jnp.where(first1, dest, 0.0), axis=-1, keepdims=True)
    d1 = jnp.sum(jnp.where(first2, dest, 0.0), axis=-1, keepdims=True)
    route_ref[...] = (jnp.where(lane == 0, d0, 0.0) + jnp.where(lane == 1, d1, 0.0)
                      + jnp.where(lane == 2, g1, 0.0) + jnp.where(lane == 3, g2v, 0.0))

    p_i = jax.lax.broadcasted_iota(jnp.int32, (T, SROWS), 1).astype(jnp.float32)
    occ = jnp.where((p_i == d0) | (p_i == d1), 1.0, 0.0)
    tv = jax.lax.broadcasted_iota(jnp.int32, (1, T), 1).astype(jnp.float32)
    src_ref[...] = jax.lax.dot_general(
        tv, occ, (((1,), (0,)), ((), ())),
        precision=jax.lax.Precision.HIGHEST,
        preferred_element_type=jnp.float32)

    blk = jax.lax.broadcasted_iota(jnp.int32, (1, 128), 1).astype(jnp.float32)
    eid = jnp.zeros((1, 128), jnp.float32) - 1.0
    for e in range(N_EXPERTS):
        be = jnp.sum(jnp.where(lane[:1] == e, base, 0.0))
        eid = eid + jnp.where(blk * BT >= be, 1.0, 0.0)
    nact = jnp.sum(cntp) * (1.0 / BT)
    meta_ref[...] = jnp.where(lane[:1] < NBLK, eid, nact)


def _grouped_moe_body(eid_ref, nact_ref, xs_ref, w1_ref, b1_ref, w2_ref,
                      b2_ref, y_ref, yacc_ref):
    f = pl.program_id(0)
    i = pl.program_id(1)
    rows = pl.ds(i * BT, BT)

    @pl.when(i < nact_ref[0])
    def _active():
        hb = jnp.maximum(
            jnp.dot(xs_ref[...], w1_ref[0], preferred_element_type=jnp.float32)
            + b1_ref[0], 0.0)
        part = jnp.dot(hb, w2_ref[0], preferred_element_type=jnp.float32)

        @pl.when(f == 0)
        def _first():
            yacc_ref[rows, :] = part

        @pl.when(f == N_FBLK - 1)
        def _last():
            y_ref[rows, :] = yacc_ref[rows, :] + part + b2_ref[0]


def _combine_body(h2_ref, route_ref, yab_ref, g3_ref, be3_ref, out_ref):
    lane = jax.lax.broadcasted_iota(jnp.int32, (T, 128), 1)
    g0 = jnp.sum(jnp.where(lane == 2, route_ref[...], 0.0), axis=-1, keepdims=True)
    g1 = jnp.sum(jnp.where(lane == 3, route_ref[...], 0.0), axis=-1, keepdims=True)
    ya = yab_ref[:T]
    yb = yab_ref[T:]
    s = h2_ref[...] + g0 * ya + g1 * yb
    out_ref[...] = _layernorm(s, g3_ref[...], be3_ref[...])


def _sc_gather(table, idx, n_rows):
    b_per_w = n_rows // NW
    mesh = plsc.VectorSubcoreMesh(core_axis_name="c", subcore_axis_name="s")

    @functools.partial(
        pl.kernel, mesh=mesh,
        out_type=jax.ShapeDtypeStruct((n_rows, D_MODEL), jnp.float32),
        scratch_types=[
            pltpu.VMEM((b_per_w,), jnp.int32),
            pltpu.VMEM((b_per_w, D_MODEL), jnp.float32),
            pltpu.SemaphoreType.DMA,
        ],
    )
    def k(table_hbm, idx_hbm, out_hbm, idx_v, rows_v, sem):
        wid = lax.axis_index("s") * NC + lax.axis_index("c")
        base = wid * b_per_w
        pltpu.sync_copy(idx_hbm.at[pl.ds(base, b_per_w)], idx_v)
        pltpu.async_copy(table_hbm.at[idx_v], rows_v, sem).wait()
        pltpu.sync_copy(rows_v, out_hbm.at[pl.ds(base, b_per_w)])

    return k(table, idx)


def kernel(x, W_emb, Wq, bq, Wk, bk, Wv, bv, Wo, bo, omega, Wg, bg,
           W1, b1, W2, b2, g2, be2, g3, be3):
    xf = x.reshape(T, -1)
    om_pad = jnp.zeros((M_PAD, D_HEAD), jnp.float32).at[:M_FEATS].set(omega)
    wg_pad = jnp.zeros((D_MODEL, 128), jnp.float32).at[:, :N_EXPERTS].set(Wg)
    bg_pad = jnp.zeros((1, 128), jnp.float32).at[0, :N_EXPERTS].set(bg)
    r2 = lambda a: a.reshape(1, -1)

    h, qn, kn, vn = pl.pallas_call(
        _embed_qkv_body,
        out_shape=[jax.ShapeDtypeStruct((T, D_MODEL), jnp.float32)] * 4,
    )(xf, W_emb, Wq, r2(bq), Wk, r2(bk), Wv, r2(bv))

    head_spec = pl.BlockSpec((1, P, HEADS_PER_BLK * D_HEAD),
                             lambda b, hh: (b, 0, hh))
    attn = pl.pallas_call(
        _performer_body,
        grid=(B, N_HEADS // HEADS_PER_BLK),
        in_specs=[head_spec, head_spec, head_spec,
                  pl.BlockSpec((M_PAD, D_HEAD), lambda b, hh: (0, 0))],
        out_specs=head_spec,
        out_shape=jax.ShapeDtypeStruct((B, P, D_MODEL), jnp.float32),
    )(qn.reshape(B, P, D_MODEL), kn.reshape(B, P, D_MODEL),
      vn.reshape(B, P, D_MODEL), om_pad)

    h2, route, srcf, meta = pl.pallas_call(
        _proj_router_body,
        out_shape=[jax.ShapeDtypeStruct((T, D_MODEL), jnp.float32),
                   jax.ShapeDtypeStruct((T, 128), jnp.float32),
                   jax.ShapeDtypeStruct((1, SROWS), jnp.float32),
                   jax.ShapeDtypeStruct((1, 128), jnp.float32)],
    )(attn.reshape(T, D_MODEL), Wo, r2(bo), h, r2(g2), r2(be2), wg_pad, bg_pad)

    src = srcf.reshape(SROWS).astype(jnp.int32)
    eid = meta[0, :NBLK].astype(jnp.int32)
    nact = meta[0, NBLK:NBLK + 1].astype(jnp.int32)
    dcat = jnp.concatenate([route[:, 0], route[:, 1]]).astype(jnp.int32)

    xs = _sc_gather(h2, src, SROWS)

    grid_spec = pltpu.PrefetchScalarGridSpec(
        num_scalar_prefetch=2,
        grid=(N_FBLK, NBLK),
        in_specs=[
            pl.BlockSpec((BT, D_MODEL), lambda f, i, eid_r, na_r: (i, 0)),
            pl.BlockSpec((1, D_MODEL, F_BLK), lambda f, i, eid_r, na_r: (eid_r[i], 0, f)),
            pl.BlockSpec((1, 1, F_BLK), lambda f, i, eid_r, na_r: (eid_r[i], 0, f)),
            pl.BlockSpec((1, F_BLK, D_MODEL), lambda f, i, eid_r, na_r: (eid_r[i], f, 0)),
            pl.BlockSpec((1, 1, D_MODEL), lambda f, i, eid_r, na_r: (eid_r[i], 0, 0)),
        ],
        out_specs=pl.BlockSpec((SROWS, D_MODEL), lambda f, i, eid_r, na_r: (0, 0)),
        scratch_shapes=[pltpu.VMEM((SROWS, D_MODEL), jnp.float32)],
    )
    y = pl.pallas_call(
        _grouped_moe_body,
        grid_spec=grid_spec,
        out_shape=jax.ShapeDtypeStruct((SROWS, D_MODEL), jnp.float32),
    )(eid, nact, xs, W1, b1.reshape(N_EXPERTS, 1, D_FF), W2,
      b2.reshape(N_EXPERTS, 1, D_MODEL))

    yab = _sc_gather(y, dcat, 2 * T)

    out = pl.pallas_call(
        _combine_body,
        out_shape=jax.ShapeDtypeStruct((T, D_MODEL), jnp.float32),
    )(h2, route, yab, r2(g3), r2(be3))

    return out.reshape(B, 1, P, D_MODEL)


# speedup vs baseline: 1.8059x; 1.8059x over previous
import functools
import math

import jax
import jax.numpy as jnp
from jax import lax
from jax.experimental import pallas as pl
from jax.experimental.pallas import tpu as pltpu
from jax.experimental.pallas import tpu_sc as plsc

B = 2
P = 512
T = B * P
D_MODEL = 768
N_HEADS = 12
D_HEAD = D_MODEL // N_HEADS
D_FF = 3072
N_EXPERTS = 4
M_FEATS = int(D_HEAD * math.log(D_HEAD))
M_PAD = 384
HEADS_PER_BLK = 2
CIN_FLAT = 128
F_BLK = 1536
N_FBLK = D_FF // F_BLK
BT = 256
NBLK = 12
SROWS = NBLK * BT
NC, NS = 2, 16
NW = NC * NS


def _embed_qkv_body(xf_ref, wemb_ref, wq_ref, bq_ref, wk_ref, bk_ref,
                    wv_ref, bv_ref, h_ref, qn_ref, kn_ref, vn_ref):
    h = jnp.dot(xf_ref[...], wemb_ref[...], preferred_element_type=jnp.float32)
    h_ref[...] = h
    dn = 1.0 / (float(D_HEAD) ** 0.25)
    qn_ref[...] = (jnp.dot(h, wq_ref[...], preferred_element_type=jnp.float32)
                   + bq_ref[...]) * dn
    kn_ref[...] = (jnp.dot(h, wk_ref[...], preferred_element_type=jnp.float32)
                   + bk_ref[...]) * dn
    vn_ref[...] = jnp.dot(h, wv_ref[...], preferred_element_type=jnp.float32) + bv_ref[...]


def _performer_one_head(qb, kb, vb, om):
    dims = (((1,), (1,)), ((), ()))
    pq = jax.lax.dot_general(qb, om, dims, preferred_element_type=jnp.float32)
    pk = jax.lax.dot_general(kb, om, dims, preferred_element_type=jnp.float32)
    lane = jax.lax.broadcasted_iota(jnp.int32, (P, M_PAD), 1)
    valid = lane < M_FEATS
    dq = 0.5 * jnp.sum(qb * qb, axis=-1, keepdims=True)
    dk = 0.5 * jnp.sum(kb * kb, axis=-1, keepdims=True)
    mq = jnp.max(pq, axis=-1, keepdims=True)
    mk = jnp.max(pk)
    inv_sqrt_m = jnp.float32(1.0 / math.sqrt(M_FEATS))
    eps = jnp.float32(1e-6)
    qp = jnp.exp(pq - (dq + mq)) * inv_sqrt_m + eps
    kp = jnp.where(valid, jnp.exp(pk - (dk + mk)) * inv_sqrt_m + eps, 0.0)
    kv = jax.lax.dot_general(kp, vb, (((0,), (0,)), ((), ())),
                             preferred_element_type=jnp.float32)
    ksum = jnp.sum(kp, axis=0, keepdims=True)
    z = jax.lax.dot_general(qp, ksum, (((1,), (1,)), ((), ())),
                            preferred_element_type=jnp.float32)
    return jnp.dot(qp, kv, preferred_element_type=jnp.float32) / z


def _performer_body(qn_ref, kn_ref, vn_ref, om_ref, o_ref):
    om = om_ref[...]
    outs = []
    for hh in range(HEADS_PER_BLK):
        sl = (slice(None), slice(hh * D_HEAD, (hh + 1) * D_HEAD))
        outs.append(_performer_one_head(
            qn_ref[0][sl], kn_ref[0][sl], vn_ref[0][sl], om))
    o_ref[0] = jnp.concatenate(outs, axis=1)


def _layernorm(s, g, b):
    mu = jnp.mean(s, axis=-1, keepdims=True)
    var = jnp.mean((s - mu) ** 2, axis=-1, keepdims=True)
    return (s - mu) * jax.lax.rsqrt(var + 1e-5) * g + b


def _proj_router_body(attn_ref, wo_ref, bo_ref, h_ref, g2_ref, be2_ref,
                      wg_ref, bg_ref, h2_ref, gates_ref):
    a = jnp.dot(attn_ref[...], wo_ref[...], preferred_element_type=jnp.float32) + bo_ref[...]
    h2 = _layernorm(h_ref[...] + a, g2_ref[...], be2_ref[...])
    h2_ref[...] = h2

    logits = jnp.dot(h2, wg_ref[...], preferred_element_type=jnp.float32) + bg_ref[...]
    lane = jax.lax.broadcasted_iota(jnp.int32, (T, 128), 1)
    logits = jnp.where(lane < N_EXPERTS, logits, jnp.float32(-1e30))
    big = jnp.int32(10**6)
    m1 = jnp.max(logits, axis=-1, keepdims=True)
    i1 = jnp.min(jnp.where(logits == m1, lane, big), axis=-1, keepdims=True)
    first1 = lane == i1
    l2 = jnp.where(first1, jnp.float32(-1e30), logits)
    m2 = jnp.max(l2, axis=-1, keepdims=True)
    i2 = jnp.min(jnp.where(l2 == m2, lane, big), axis=-1, keepdims=True)
    first2 = lane == i2
    ex = jnp.exp(m2 - m1)
    g1 = 1.0 / (1.0 + ex)
    g2v = ex / (1.0 + ex)
    gates_ref[...] = jnp.where(first1, g1, 0.0) + jnp.where(first2, g2v, 0.0)


def _moe_dense_body(h2_ref, gates_ref, w1_ref, b1_ref, w2_ref, b2_ref,
                    g3_ref, be3_ref, out_ref, acc_ref, eacc_ref):
    e = pl.program_id(0)
    f = pl.program_id(1)

    @pl.when((e == 0) & (f == 0))
    def _init():
        acc_ref[...] = h2_ref[...]

    hb = jnp.maximum(
        jnp.dot(h2_ref[...], w1_ref[0], preferred_element_type=jnp.float32)
        + b1_ref[0], 0.0)
    part = jnp.dot(hb, w2_ref[0], preferred_element_type=jnp.float32)

    @pl.when(f == 0)
    def _estart():
        eacc_ref[...] = part

    @pl.when(f == N_FBLK - 1)
    def _efin():
        lane = jax.lax.broadcasted_iota(jnp.int32, (T, 128), 1)
        ge = jnp.sum(jnp.where(lane == e, gates_ref[...], 0.0),
                     axis=-1, keepdims=True)
        acc_ref[...] += ge * (eacc_ref[...] + part + b2_ref[0])

    @pl.when((e == N_EXPERTS - 1) & (f == N_FBLK - 1))
    def _final():
        out_ref[...] = _layernorm(acc_ref[...], g3_ref[...], be3_ref[...])


def kernel(x, W_emb, Wq, bq, Wk, bk, Wv, bv, Wo, bo, omega, Wg, bg,
           W1, b1, W2, b2, g2, be2, g3, be3):
    xf = x.reshape(T, -1)
    om_pad = jnp.zeros((M_PAD, D_HEAD), jnp.float32).at[:M_FEATS].set(omega)
    wg_pad = jnp.zeros((D_MODEL, 128), jnp.float32).at[:, :N_EXPERTS].set(Wg)
    bg_pad = jnp.zeros((1, 128), jnp.float32).at[0, :N_EXPERTS].set(bg)
    r2 = lambda a: a.reshape(1, -1)

    h, qn, kn, vn = pl.pallas_call(
        _embed_qkv_body,
        out_shape=[jax.ShapeDtypeStruct((T, D_MODEL), jnp.float32)] * 4,
    )(xf, W_emb, Wq, r2(bq), Wk, r2(bk), Wv, r2(bv))

    head_spec = pl.BlockSpec((1, P, HEADS_PER_BLK * D_HEAD),
                             lambda b, hh: (b, 0, hh))
    attn = pl.pallas_call(
        _performer_body,
        grid=(B, N_HEADS // HEADS_PER_BLK),
        in_specs=[head_spec, head_spec, head_spec,
                  pl.BlockSpec((M_PAD, D_HEAD), lambda b, hh: (0, 0))],
        out_specs=head_spec,
        out_shape=jax.ShapeDtypeStruct((B, P, D_MODEL), jnp.float32),
    )(qn.reshape(B, P, D_MODEL), kn.reshape(B, P, D_MODEL),
      vn.reshape(B, P, D_MODEL), om_pad)

    h2, gates = pl.pallas_call(
        _proj_router_body,
        out_shape=[jax.ShapeDtypeStruct((T, D_MODEL), jnp.float32),
                   jax.ShapeDtypeStruct((T, 128), jnp.float32)],
    )(attn.reshape(T, D_MODEL), Wo, r2(bo), h, r2(g2), r2(be2), wg_pad, bg_pad)

    full = lambda shape: pl.BlockSpec(shape, lambda e, f: (0,) * len(shape))
    out = pl.pallas_call(
        _moe_dense_body,
        grid=(N_EXPERTS, N_FBLK),
        in_specs=[
            full((T, D_MODEL)),
            full((T, 128)),
            pl.BlockSpec((1, D_MODEL, F_BLK), lambda e, f: (e, 0, f)),
            pl.BlockSpec((1, 1, F_BLK), lambda e, f: (e, 0, f)),
            pl.BlockSpec((1, F_BLK, D_MODEL), lambda e, f: (e, f, 0)),
            pl.BlockSpec((1, 1, D_MODEL), lambda e, f: (e, 0, 0)),
            full((1, D_MODEL)),
            full((1, D_MODEL)),
        ],
        out_specs=full((T, D_MODEL)),
        out_shape=jax.ShapeDtypeStruct((T, D_MODEL), jnp.float32),
        scratch_shapes=[pltpu.VMEM((T, D_MODEL), jnp.float32),
                        pltpu.VMEM((T, D_MODEL), jnp.float32)],
    )(h2, gates, W1, b1.reshape(N_EXPERTS, 1, D_FF), W2,
      b2.reshape(N_EXPERTS, 1, D_MODEL), r2(g3), r2(be3))

    return out.reshape(B, 1, P, D_MODEL)
